# baseline (speedup 1.0000x reference)
import functools

import jax
import jax.numpy as jnp
from jax import lax
from jax.experimental import pallas as pl
from jax.experimental.pallas import tpu as pltpu
from jax.experimental.pallas import tpu_sc as plsc

_VOCAB = 100000
_EMB = 16
_BATCH = 1024
_CTX = 20

_NC = 2
_NS = 16
_NW = _NC * _NS
_BPW = _BATCH // _NW
_IPW = _BPW * _CTX
_CHUNK = 128
_NCHUNK = _IPW // _CHUNK

_BN = 1024


def _gather_sum_sc(x3, table):
    mesh = plsc.VectorSubcoreMesh(core_axis_name="c", subcore_axis_name="s")

    @functools.partial(
        pl.kernel,
        mesh=mesh,
        out_type=jax.ShapeDtypeStruct((_BATCH, _EMB), jnp.float32),
        scratch_types=[
            pltpu.VMEM((_NCHUNK, _CHUNK), jnp.int32),
            pltpu.VMEM((_IPW, _EMB), jnp.float32),
            pltpu.VMEM((_BPW, _EMB), jnp.float32),
            pltpu.SemaphoreType.DMA,
        ],
        compiler_params=pltpu.CompilerParams(use_tc_tiling_on_sc=False),
    )
    def run(x_hbm, tab_hbm, out_hbm, idx_v, rows_v, acc_v, sem):
        wid = lax.axis_index("s") * _NC + lax.axis_index("c")
        pltpu.sync_copy(x_hbm.at[wid], idx_v)
        copies = []
        for c in range(_NCHUNK):
            copies.append(
                pltpu.async_copy(
                    tab_hbm.at[idx_v.at[c]],
                    rows_v.at[pl.ds(c * _CHUNK, _CHUNK)],
                    sem,
                )
            )
        for cp in copies:
            cp.wait()
        for j in range(_BPW):
            acc = rows_v[j * _CTX, :]
            for t in range(1, _CTX):
                acc = acc + rows_v[j * _CTX + t, :]
            acc_v[j, :] = acc
        pltpu.sync_copy(acc_v, out_hbm.at[pl.ds(wid * _BPW, _BPW)])

    return run(x3, table)


def _project_tc(sT, Wt, b2):
    nb = pl.cdiv(_VOCAB, _BN)

    def body(w_ref, s_ref, b_ref, o_ref):
        w_aug = jnp.concatenate([w_ref[...], b_ref[...]], axis=0)
        s_aug = jnp.concatenate(
            [s_ref[...], jnp.ones((1, _BATCH), jnp.float32)], axis=0
        )
        o_ref[...] = lax.dot_general(
            w_aug,
            s_aug,
            (((0,), (0,)), ((), ())),
            preferred_element_type=jnp.float32,
        )

    return pl.pallas_call(
        body,
        grid=(nb,),
        in_specs=[
            pl.BlockSpec((_EMB, _BN), lambda j: (0, j)),
            pl.BlockSpec((_EMB, _BATCH), lambda j: (0, 0)),
            pl.BlockSpec((1, _BN), lambda j: (0, j)),
        ],
        out_specs=pl.BlockSpec((_BN, _BATCH), lambda j: (j, 0)),
        out_shape=jax.ShapeDtypeStruct((_VOCAB, _BATCH), jnp.float32),
    )(Wt, sT, b2)


def kernel(x, embedding_matrix, W, b):
    x3 = x.reshape(_NW, _NCHUNK, _CHUNK)
    summed = _gather_sum_sc(x3, embedding_matrix)
    outT = _project_tc(summed.T, W.T, b.reshape(1, _VOCAB))
    return outT.T


# speedup vs baseline: 1.2088x; 1.2088x over previous
import functools

import jax
import jax.numpy as jnp
from jax import lax
from jax.experimental import pallas as pl
from jax.experimental.pallas import tpu as pltpu
from jax.experimental.pallas import tpu_sc as plsc

_VOCAB = 100000
_EMB = 16
_BATCH = 1024
_CTX = 20

_NC = 2
_NS = 16
_NW = _NC * _NS
_BPW = _BATCH // _NW
_IPW = _BPW * _CTX
_CHUNK = 128
_NCHUNK = _IPW // _CHUNK

_BN = 2048


def _gather_sum_sc(x2, tabT):
    mesh = plsc.VectorSubcoreMesh(core_axis_name="c", subcore_axis_name="s")

    @functools.partial(
        pl.kernel,
        mesh=mesh,
        out_type=jax.ShapeDtypeStruct((_BATCH, _EMB), jnp.float32),
        scratch_types=[
            pltpu.VMEM((_IPW,), jnp.int32),
            pltpu.VMEM((_EMB, _IPW), jnp.int32),
            pltpu.VMEM((_EMB, _IPW), jnp.float32),
            pltpu.VMEM((_BPW, _EMB), jnp.float32),
            pltpu.SemaphoreType.DMA,
        ],
        compiler_params=pltpu.CompilerParams(
            use_tc_tiling_on_sc=False, needs_layout_passes=False
        ),
    )
    def run(x_hbm, tab_hbm, out_hbm, idx_v, pidx_v, vals_v, acc_v, sem):
        wid = lax.axis_index("s") * _NC + lax.axis_index("c")
        pltpu.sync_copy(x_hbm.at[wid], idx_v)
        for k in range(_IPW // 16):
            v = idx_v[pl.ds(k * 16, 16)]
            for e in range(_EMB):
                pidx_v[e, pl.ds(k * 16, 16)] = v + (e * _VOCAB)
        copies = []
        for e in range(_EMB):
            for c in range(_NCHUNK):
                copies.append(
                    pltpu.async_copy(
                        tab_hbm.at[pidx_v.at[e, pl.ds(c * _CHUNK, _CHUNK)]],
                        vals_v.at[e, pl.ds(c * _CHUNK, _CHUNK)],
                        sem,
                    )
                )
        for cp in copies:
            cp.wait()
        lane = lax.iota(jnp.int32, 16)

        def body(j, _):
            acc = jnp.zeros((16,), jnp.float32)
            for t in range(_CTX):
                iv = jnp.full((16,), j * _CTX + t, jnp.int32)
                acc = acc + plsc.load_gather(vals_v, [lane, iv])
            acc_v[j, :] = acc
            return _

        lax.fori_loop(0, _BPW, body, None)
        pltpu.sync_copy(acc_v, out_hbm.at[pl.ds(wid * _BPW, _BPW)])

    return run(x2, tabT)


def _project_tc(sT, Wt, b2):
    nb = pl.cdiv(_VOCAB, _BN)

    def body(w_ref, s_ref, b_ref, o_ref):
        w_aug = jnp.concatenate([w_ref[...], b_ref[...]], axis=0)
        s_aug = jnp.concatenate(
            [s_ref[...], jnp.ones((1, _BATCH), jnp.float32)], axis=0
        )
        o_ref[...] = lax.dot_general(
            w_aug,
            s_aug,
            (((0,), (0,)), ((), ())),
            preferred_element_type=jnp.float32,
        )

    return pl.pallas_call(
        body,
        grid=(nb,),
        in_specs=[
            pl.BlockSpec((_EMB, _BN), lambda j: (0, j)),
            pl.BlockSpec((_EMB, _BATCH), lambda j: (0, 0)),
            pl.BlockSpec((1, _BN), lambda j: (0, j)),
        ],
        out_specs=pl.BlockSpec((_BN, _BATCH), lambda j: (j, 0)),
        out_shape=jax.ShapeDtypeStruct((_VOCAB, _BATCH), jnp.float32),
    )(Wt, sT, b2)


def kernel(x, embedding_matrix, W, b):
    x2 = x.reshape(_NW, _IPW)
    tabT = embedding_matrix.T.reshape(_EMB * _VOCAB)
    summed = _gather_sum_sc(x2, tabT)
    outT = _project_tc(summed.T, W.T, b.reshape(1, _VOCAB))
    return outT.T


# speedup vs baseline: 1.2382x; 1.0243x over previous
import functools

import jax
import jax.numpy as jnp
from jax import lax
from jax.experimental import pallas as pl
from jax.experimental.pallas import tpu as pltpu
from jax.experimental.pallas import tpu_sc as plsc

_VOCAB = 100000
_EMB = 16
_BATCH = 1024
_CTX = 20

_NC = 2
_NS = 16
_NW = _NC * _NS
_BPW = _BATCH // _NW
_IPW = _BPW * _CTX
_CHUNK = 128
_NCHUNK = _IPW // _CHUNK
_DEPTH = 4

_BN = 2048


def _gather_sum_sc(x2, tabT):
    mesh = plsc.VectorSubcoreMesh(core_axis_name="c", subcore_axis_name="s")

    @functools.partial(
        pl.kernel,
        mesh=mesh,
        out_type=jax.ShapeDtypeStruct((_EMB, _BATCH), jnp.float32),
        scratch_types=[
            pltpu.VMEM((_IPW,), jnp.int32),
            pltpu.VMEM((_EMB, _IPW), jnp.int32),
            pltpu.VMEM((_EMB, _IPW), jnp.float32),
            pltpu.VMEM((_EMB, _BPW), jnp.float32),
            pltpu.SemaphoreType.DMA((_EMB,)),
        ],
        compiler_params=pltpu.CompilerParams(
            use_tc_tiling_on_sc=False, needs_layout_passes=False
        ),
    )
    def run(x_hbm, tab_hbm, out_hbm, idx_v, pidx_v, vals_v, accT_v, sem):
        wid = lax.axis_index("s") * _NC + lax.axis_index("c")
        pltpu.sync_copy(x_hbm.at[wid], idx_v)
        lane = lax.iota(jnp.int32, 16)
        copies = {}
        for step in range(_EMB + _DEPTH):
            if step < _EMB:
                e = step
                for k in range(_IPW // 16):
                    pidx_v[e, pl.ds(k * 16, 16)] = (
                        idx_v[pl.ds(k * 16, 16)] + (e * _VOCAB)
                    )
                copies[e] = [
                    pltpu.async_copy(
                        tab_hbm.at[pidx_v.at[e, pl.ds(c * _CHUNK, _CHUNK)]],
                        vals_v.at[e, pl.ds(c * _CHUNK, _CHUNK)],
                        sem.at[e],
                    )
                    for c in range(_NCHUNK)
                ]
            if step >= _DEPTH:
                e = step - _DEPTH
                for cp in copies.pop(e):
                    cp.wait()
                for g in range(0, _BPW, 16):
                    jbase = (lane + g) * _CTX
                    acc = jnp.zeros((16,), jnp.float32)
                    for t in range(_CTX):
                        ev = jnp.full((16,), e, jnp.int32)
                        acc = acc + plsc.load_gather(vals_v, [ev, jbase + t])
                    accT_v[e, pl.ds(g, 16)] = acc
        pltpu.sync_copy(accT_v, out_hbm.at[:, pl.ds(wid * _BPW, _BPW)])

    return run(x2, tabT)


def _project_tc(sT, Wt, b2):
    nb = pl.cdiv(_VOCAB, _BN)

    def body(w_ref, s_ref, b_ref, o_ref):
        w_aug = jnp.concatenate([w_ref[...], b_ref[...]], axis=0)
        s_aug = jnp.concatenate(
            [s_ref[...], jnp.ones((1, _BATCH), jnp.float32)], axis=0
        )
        o_ref[...] = lax.dot_general(
            w_aug,
            s_aug,
            (((0,), (0,)), ((), ())),
            preferred_element_type=jnp.float32,
        )

    return pl.pallas_call(
        body,
        grid=(nb,),
        in_specs=[
            pl.BlockSpec((_EMB, _BN), lambda j: (0, j)),
            pl.BlockSpec((_EMB, _BATCH), lambda j: (0, 0)),
            pl.BlockSpec((1, _BN), lambda j: (0, j)),
        ],
        out_specs=pl.BlockSpec((_BN, _BATCH), lambda j: (j, 0)),
        out_shape=jax.ShapeDtypeStruct((_VOCAB, _BATCH), jnp.float32),
    )(Wt, sT, b2)


def kernel(x, embedding_matrix, W, b):
    x2 = x.reshape(_NW, _IPW)
    tabT = embedding_matrix.T.reshape(_EMB * _VOCAB)
    sT = _gather_sum_sc(x2, tabT)
    outT = _project_tc(sT, W.T, b.reshape(1, _VOCAB))
    return outT.T


# speedup vs baseline: 1.2390x; 1.0007x over previous
import functools

import jax
import jax.numpy as jnp
from jax import lax
from jax.experimental import pallas as pl
from jax.experimental.pallas import tpu as pltpu
from jax.experimental.pallas import tpu_sc as plsc

_VOCAB = 100000
_EMB = 16
_BATCH = 1024
_CTX = 20

_NC = 2
_NS = 16
_NW = _NC * _NS
_BPW = _BATCH // _NW
_IPW = _BPW * _CTX
_CHUNK = 128
_NCHUNK = _IPW // _CHUNK
_DEPTH = 8

_BN = 2048


def _gather_sum_sc(x2, tabT):
    mesh = plsc.VectorSubcoreMesh(core_axis_name="c", subcore_axis_name="s")

    @functools.partial(
        pl.kernel,
        mesh=mesh,
        out_type=jax.ShapeDtypeStruct((_EMB, _BATCH), jnp.float32),
        scratch_types=[
            pltpu.VMEM((_IPW,), jnp.int32),
            pltpu.VMEM((_EMB, _IPW), jnp.int32),
            pltpu.VMEM((_EMB, _IPW), jnp.float32),
            pltpu.VMEM((_EMB, _BPW), jnp.float32),
            pltpu.SemaphoreType.DMA((_EMB,)),
        ],
        compiler_params=pltpu.CompilerParams(
            use_tc_tiling_on_sc=False, needs_layout_passes=False
        ),
    )
    def run(x_hbm, tab_hbm, out_hbm, idx_v, pidx_v, vals_v, accT_v, sem):
        wid = lax.axis_index("s") * _NC + lax.axis_index("c")
        pltpu.sync_copy(x_hbm.at[wid], idx_v)
        lane = lax.iota(jnp.int32, 16)
        copies = {}
        for step in range(_EMB + _DEPTH):
            if step < _EMB:
                e = step
                for k in range(_IPW // 16):
                    pidx_v[e, pl.ds(k * 16, 16)] = (
                        idx_v[pl.ds(k * 16, 16)] + (e * _VOCAB)
                    )
                copies[e] = [
                    pltpu.async_copy(
                        tab_hbm.at[pidx_v.at[e, pl.ds(c * _CHUNK, _CHUNK)]],
                        vals_v.at[e, pl.ds(c * _CHUNK, _CHUNK)],
                        sem.at[e],
                    )
                    for c in range(_NCHUNK)
                ]
            if step >= _DEPTH:
                e = step - _DEPTH
                for cp in copies.pop(e):
                    cp.wait()
                for g in range(0, _BPW, 16):
                    jbase = (lane + g) * _CTX
                    acc = jnp.zeros((16,), jnp.float32)
                    for t in range(_CTX):
                        ev = jnp.full((16,), e, jnp.int32)
                        acc = acc + plsc.load_gather(vals_v, [ev, jbase + t])
                    accT_v[e, pl.ds(g, 16)] = acc
        pltpu.sync_copy(accT_v, out_hbm.at[:, pl.ds(wid * _BPW, _BPW)])

    return run(x2, tabT)


def _project_tc(sT, Wt, b2):
    nb = pl.cdiv(_VOCAB, _BN)

    def body(w_ref, s_ref, b_ref, o_ref):
        w_aug = jnp.concatenate([w_ref[...], b_ref[...]], axis=0)
        s_aug = jnp.concatenate(
            [s_ref[...], jnp.ones((1, _BATCH), jnp.float32)], axis=0
        )
        o_ref[...] = lax.dot_general(
            w_aug,
            s_aug,
            (((0,), (0,)), ((), ())),
            preferred_element_type=jnp.float32,
        )

    return pl.pallas_call(
        body,
        grid=(nb,),
        in_specs=[
            pl.BlockSpec((_EMB, _BN), lambda j: (0, j)),
            pl.BlockSpec((_EMB, _BATCH), lambda j: (0, 0)),
            pl.BlockSpec((1, _BN), lambda j: (0, j)),
        ],
        out_specs=pl.BlockSpec((_BN, _BATCH), lambda j: (j, 0)),
        out_shape=jax.ShapeDtypeStruct((_VOCAB, _BATCH), jnp.float32),
    )(Wt, sT, b2)


def kernel(x, embedding_matrix, W, b):
    x2 = x.reshape(_NW, _IPW)
    tabT = embedding_matrix.T.reshape(_EMB * _VOCAB)
    sT = _gather_sum_sc(x2, tabT)
    outT = _project_tc(sT, W.T, b.reshape(1, _VOCAB))
    return outT.T
